# baseline (speedup 1.0000x reference)
import functools

import jax
import jax.numpy as jnp
from jax import lax
from jax.experimental import pallas as pl
from jax.experimental.pallas import tpu as pltpu
from jax.experimental.pallas import tpu_sc as plsc

N_NODES = 10000
N_PAD = 10240
N_EDGES = 320000
D = 128
DH = D // 2
NC = 2
NS = 16
NW = NC * NS
CH = 80
NCH_DEG = N_EDGES // NW // CH
NCH_AGG = N_EDGES // NS // CH
RT = N_PAD // NS
DW = 16
NB = 4
NR_DEG = (NCH_DEG - NB) // NB
NR_AGG = (NCH_AGG - NB) // NB

_mesh = plsc.VectorSubcoreMesh(core_axis_name="c", subcore_axis_name="s")


@functools.partial(
    pl.kernel,
    out_type=jax.ShapeDtypeStruct((NC * N_PAD, DW), jnp.float32),
    mesh=_mesh,
    scratch_types=[
        pltpu.VMEM((CH,), jnp.int32),
        pltpu.VMEM((CH, DW), jnp.float32),
        pltpu.VMEM((CH, DW), jnp.float32),
        pltpu.VMEM_SHARED((N_PAD, DW), jnp.float32),
    ],
)
def _sc_deg(dst_hbm, deg_hbm, idx_v, ones_v, zeros_v, acc):
    c = lax.axis_index("c")
    s = lax.axis_index("s")
    w = s * NC + c

    one = jnp.full((16,), 1.0, jnp.float32)
    z = jnp.zeros((16,), jnp.float32)

    def fill(i, _):
        ones_v[i, pl.ds(0, 16)] = one
        zeros_v[i, pl.ds(0, 16)] = z
        return 0

    lax.fori_loop(0, CH, fill, 0)

    def zinit(i, _):
        pltpu.sync_copy(zeros_v, acc.at[pl.ds(s * RT + i * CH, CH)])
        return 0

    lax.fori_loop(0, RT // CH, zinit, 0)
    plsc.subcore_barrier()

    def edge_body(g, _):
        pltpu.sync_copy(dst_hbm.at[w, g], idx_v)
        pltpu.sync_copy(ones_v, acc.at[idx_v], add=True)
        return 0

    lax.fori_loop(0, NCH_DEG, edge_body, 0)
    plsc.subcore_barrier()

    pltpu.sync_copy(acc.at[pl.ds(s * RT, RT)],
                    deg_hbm.at[pl.ds(c * N_PAD + s * RT, RT)])


@functools.partial(
    pl.kernel,
    out_type=jax.ShapeDtypeStruct((N_PAD, D), jnp.float32),
    mesh=_mesh,
    scratch_types=[
        pltpu.VMEM((NCH_AGG, CH), jnp.int32),
        pltpu.VMEM((NB, CH, DH), jnp.float32),
        pltpu.VMEM_SHARED((N_PAD, DH), jnp.float32),
        pltpu.SemaphoreType.DMA((NB,)),
        pltpu.SemaphoreType.DMA((NB,)),
        pltpu.SemaphoreType.DMA((NB,)),
        pltpu.VMEM((CH,), jnp.int32),
        pltpu.VMEM((CH,), jnp.int32),
        pltpu.VMEM((CH,), jnp.int32),
        pltpu.VMEM((CH,), jnp.int32),
    ],
    compiler_params=pltpu.CompilerParams(use_tc_tiling_on_sc=False),
)
def _sc_agg(xsp_hbm, srcAB_hbm, dst_hbm, out_hbm,
            src_all, rows_v, acc, sem_g, sem_s, sem_i,
            dst_v0, dst_v1, dst_v2, dst_v3):
    dst_v = (dst_v0, dst_v1, dst_v2, dst_v3)
    c = lax.axis_index("c")
    s = lax.axis_index("s")
    w = c * NS + s

    pltpu.sync_copy(srcAB_hbm.at[w], src_all)

    z = jnp.zeros((16,), jnp.float32)

    def zrow(i, _):
        for j in range(DH // 16):
            rows_v[0, i, pl.ds(j * 16, 16)] = z
        return 0

    lax.fori_loop(0, CH, zrow, 0)

    def zinit(i, _):
        pltpu.sync_copy(rows_v.at[0], acc.at[pl.ds(s * RT + i * CH, CH)])
        return 0

    lax.fori_loop(0, RT // CH, zinit, 0)

    for j in range(NB):
        pltpu.sync_copy(dst_hbm.at[s, j], dst_v[j])
        pltpu.async_copy(xsp_hbm.at[src_all.at[j]], rows_v.at[j],
                         sem_g.at[j])

    plsc.subcore_barrier()

    def round_body(r, _):
        base = r * NB
        for j in range(NB):
            g = base + j
            pltpu.make_async_copy(xsp_hbm.at[src_all.at[g]], rows_v.at[j],
                                  sem_g.at[j]).wait()
            pltpu.async_copy(rows_v.at[j], acc.at[dst_v[j]],
                             sem_s.at[j], add=True)
        for j in range(NB):
            g = base + j
            pltpu.make_async_copy(rows_v.at[j], acc.at[dst_v[j]],
                                  sem_s.at[j]).wait()
            pltpu.sync_copy(dst_hbm.at[s, g + NB], dst_v[j])
            pltpu.async_copy(xsp_hbm.at[src_all.at[g + NB]], rows_v.at[j],
                             sem_g.at[j])
        return 0

    lax.fori_loop(0, NR_AGG, round_body, 0)

    for g in range(NR_AGG * NB, NCH_AGG):
        j = g % NB
        pltpu.make_async_copy(xsp_hbm.at[src_all.at[g]], rows_v.at[j],
                              sem_g.at[j]).wait()
        pltpu.sync_copy(rows_v.at[j], acc.at[dst_v[j]], add=True)
        if g + NB < NCH_AGG:
            pltpu.sync_copy(dst_hbm.at[s, g + NB], dst_v[j])
            pltpu.async_copy(xsp_hbm.at[src_all.at[g + NB]], rows_v.at[j],
                             sem_g.at[j])

    plsc.subcore_barrier()

    pltpu.sync_copy(acc.at[pl.ds(s * RT, RT)],
                    out_hbm.at[pl.ds(s * RT, RT), pl.ds(c * DH, DH)])


_BLK = 512
_GRID = N_PAD // _BLK
_full = pl.BlockSpec((_BLK, D), lambda i: (i, 0))
_col = pl.BlockSpec((_BLK, 1), lambda i: (i, 0))


def _tc_mm_body(x_ref, w_ref, deg_ref, o_ref, d_ref):
    dg = deg_ref[...]
    dv = lax.rsqrt(dg[0, :, 0:1] + dg[1, :, 0:1] + 1.0)
    y = lax.dot_general(x_ref[...], w_ref[...], (((1,), (1,)), ((), ())),
                        preferred_element_type=jnp.float32)
    o_ref[...] = y * dv
    d_ref[...] = dv


def _tc_mm(x, w, deg2):
    return pl.pallas_call(
        _tc_mm_body,
        grid=(_GRID,),
        in_specs=[
            _full,
            pl.BlockSpec((D, D), lambda i: (0, 0)),
            pl.BlockSpec((2, _BLK, DW), lambda i: (0, i, 0)),
        ],
        out_specs=[_full, _col],
        out_shape=[jax.ShapeDtypeStruct((N_PAD, D), jnp.float32),
                   jax.ShapeDtypeStruct((N_PAD, 1), jnp.float32)],
    )(x, w, deg2)


def _tc_fuse_body(p_ref, xs_ref, d_ref, b_ref, w_ref, o_ref):
    dv = d_ref[...]
    h = jnp.maximum((p_ref[...] + xs_ref[...]) * dv + b_ref[...], 0.0)
    y = lax.dot_general(h, w_ref[...], (((1,), (1,)), ((), ())),
                        preferred_element_type=jnp.float32)
    o_ref[...] = y * dv


def _tc_fuse(p, xs, dinv, b, w):
    return pl.pallas_call(
        _tc_fuse_body,
        grid=(_GRID,),
        in_specs=[
            _full, _full, _col,
            pl.BlockSpec((1, D), lambda i: (0, 0)),
            pl.BlockSpec((D, D), lambda i: (0, 0)),
        ],
        out_specs=_full,
        out_shape=jax.ShapeDtypeStruct((N_PAD, D), jnp.float32),
    )(p, xs, dinv, b, w)


def _tc_comb_body(p_ref, xs_ref, d_ref, b_ref, o_ref):
    o_ref[...] = jnp.maximum(
        (p_ref[...] + xs_ref[...]) * d_ref[...] + b_ref[...], 0.0)


def _tc_comb(p, xs, dinv, b):
    return pl.pallas_call(
        _tc_comb_body,
        grid=(_GRID,),
        in_specs=[_full, _full, _col,
                  pl.BlockSpec((1, D), lambda i: (0, 0))],
        out_specs=_full,
        out_shape=jax.ShapeDtypeStruct((N_PAD, D), jnp.float32),
    )(p, xs, dinv, b)


def kernel(x, edge_index, W1, b1, W2, b2):
    src = edge_index[0].astype(jnp.int32)
    dst = edge_index[1].astype(jnp.int32)
    x_p = jnp.pad(x, ((0, N_PAD - N_NODES), (0, 0)))
    b1r = b1.reshape(1, D)
    b2r = b2.reshape(1, D)
    s2 = 2 * src
    srcAB = jnp.concatenate([s2, s2 + 1]).reshape(NC * NS, NCH_AGG, CH)
    dstD = dst.reshape(NW, NCH_DEG, CH)
    dstA = dst.reshape(NS, NCH_AGG, CH)

    deg2 = _sc_deg(dstD).reshape(NC, N_PAD, DW)
    xs1, dinv = _tc_mm(x_p, W1, deg2)

    p1 = _sc_agg(xs1.reshape(2 * N_PAD, DH), srcAB, dstA)
    xs2 = _tc_fuse(p1, xs1, dinv, b1r, W2)
    p2 = _sc_agg(xs2.reshape(2 * N_PAD, DH), srcAB, dstA)
    h2 = _tc_comb(p2, xs2, dinv, b2r)
    return h2[:N_NODES]


# speedup vs baseline: 1.3262x; 1.3262x over previous
import functools

import jax
import jax.numpy as jnp
from jax import lax
from jax.experimental import pallas as pl
from jax.experimental.pallas import tpu as pltpu
from jax.experimental.pallas import tpu_sc as plsc

N_NODES = 10000
N_PAD = 10240
N_EDGES = 320000
D = 128
DH = D // 2
NC = 2
NS = 16
NW = NC * NS
CH = 80
NCH_DEG = N_EDGES // NW // CH
NCH_AGG = N_EDGES // NS // CH
RT = N_PAD // NS
DW = 16
NB = 4
NR_DEG = (NCH_DEG - NB) // NB
NR_AGG = (NCH_AGG - NB) // NB

_mesh = plsc.VectorSubcoreMesh(core_axis_name="c", subcore_axis_name="s")


@functools.partial(
    pl.kernel,
    out_type=jax.ShapeDtypeStruct((NC * N_PAD, DW), jnp.float32),
    mesh=_mesh,
    scratch_types=[
        pltpu.VMEM((CH,), jnp.int32),
        pltpu.VMEM((CH, DW), jnp.float32),
        pltpu.VMEM((CH, DW), jnp.float32),
        pltpu.VMEM_SHARED((N_PAD, DW), jnp.float32),
    ],
)
def _sc_deg(dst_hbm, deg_hbm, idx_v, ones_v, zeros_v, acc):
    c = lax.axis_index("c")
    s = lax.axis_index("s")
    w = s * NC + c

    one = jnp.full((16,), 1.0, jnp.float32)
    z = jnp.zeros((16,), jnp.float32)

    def fill(i, _):
        ones_v[i, pl.ds(0, 16)] = one
        zeros_v[i, pl.ds(0, 16)] = z
        return 0

    lax.fori_loop(0, CH, fill, 0)

    def zinit(i, _):
        pltpu.sync_copy(zeros_v, acc.at[pl.ds(s * RT + i * CH, CH)])
        return 0

    lax.fori_loop(0, RT // CH, zinit, 0)
    plsc.subcore_barrier()

    def edge_body(g, _):
        pltpu.sync_copy(dst_hbm.at[w, g], idx_v)
        pltpu.sync_copy(ones_v, acc.at[idx_v], add=True)
        return 0

    lax.fori_loop(0, NCH_DEG, edge_body, 0)
    plsc.subcore_barrier()

    pltpu.sync_copy(acc.at[pl.ds(s * RT, RT)],
                    deg_hbm.at[pl.ds(c * N_PAD + s * RT, RT)])


@functools.partial(
    pl.kernel,
    out_type=jax.ShapeDtypeStruct((N_PAD, D), jnp.float32),
    mesh=_mesh,
    scratch_types=[
        pltpu.VMEM((NCH_AGG, CH), jnp.int32),
        pltpu.VMEM((NB, CH, DH), jnp.float32),
        pltpu.VMEM_SHARED((N_PAD, DH), jnp.float32),
        pltpu.SemaphoreType.DMA((NB,)),
        pltpu.SemaphoreType.DMA((NB,)),
        pltpu.SemaphoreType.DMA((NB,)),
        pltpu.VMEM((CH,), jnp.int32),
        pltpu.VMEM((CH,), jnp.int32),
        pltpu.VMEM((CH,), jnp.int32),
        pltpu.VMEM((CH,), jnp.int32),
    ],
    compiler_params=pltpu.CompilerParams(use_tc_tiling_on_sc=False),
)
def _sc_agg(xsp_hbm, srcAB_hbm, dst_hbm, out_hbm,
            src_all, rows_v, acc, sem_g, sem_s, sem_i,
            dst_v0, dst_v1, dst_v2, dst_v3):
    dst_v = (dst_v0, dst_v1, dst_v2, dst_v3)
    c = lax.axis_index("c")
    s = lax.axis_index("s")
    w = c * NS + s

    pltpu.sync_copy(srcAB_hbm.at[w], src_all)

    z = jnp.zeros((16,), jnp.float32)

    def zrow(i, _):
        for j in range(DH // 16):
            rows_v[0, i, pl.ds(j * 16, 16)] = z
        return 0

    lax.fori_loop(0, CH, zrow, 0)

    def zinit(i, _):
        pltpu.sync_copy(rows_v.at[0], acc.at[pl.ds(s * RT + i * CH, CH)])
        return 0

    lax.fori_loop(0, RT // CH, zinit, 0)

    for j in range(NB):
        pltpu.async_copy(dst_hbm.at[s * NCH_AGG + j], dst_v[j], sem_i.at[j])
        pltpu.async_copy(xsp_hbm.at[src_all.at[j]], rows_v.at[j],
                         sem_g.at[j])

    plsc.subcore_barrier()

    def round_body(r, _):
        base = r * NB
        for j in range(NB):
            g = base + j
            pltpu.make_async_copy(xsp_hbm.at[src_all.at[g]], rows_v.at[j],
                                  sem_g.at[j]).wait()
            pltpu.make_async_copy(dst_hbm.at[s * NCH_AGG + g],
                                  dst_v[j], sem_i.at[j]).wait()
            pltpu.async_copy(rows_v.at[j], acc.at[dst_v[j]],
                             sem_s.at[j], add=True)
        for j in range(NB):
            g = base + j
            pltpu.make_async_copy(rows_v.at[j], acc.at[dst_v[j]],
                                  sem_s.at[j]).wait()
            pltpu.async_copy(dst_hbm.at[s * NCH_AGG + g + NB], dst_v[j], sem_i.at[j])
            pltpu.async_copy(xsp_hbm.at[src_all.at[g + NB]], rows_v.at[j],
                             sem_g.at[j])
        return 0

    lax.fori_loop(0, NR_AGG, round_body, 0)

    for g in range(NR_AGG * NB, NCH_AGG):
        j = g % NB
        pltpu.make_async_copy(xsp_hbm.at[src_all.at[g]], rows_v.at[j],
                              sem_g.at[j]).wait()
        pltpu.make_async_copy(dst_hbm.at[s * NCH_AGG + g], dst_v[j], sem_i.at[j]).wait()
        pltpu.sync_copy(rows_v.at[j], acc.at[dst_v[j]], add=True)
        if g + NB < NCH_AGG:
            pltpu.async_copy(dst_hbm.at[s * NCH_AGG + g + NB], dst_v[j], sem_i.at[j])
            pltpu.async_copy(xsp_hbm.at[src_all.at[g + NB]], rows_v.at[j],
                             sem_g.at[j])

    plsc.subcore_barrier()

    pltpu.sync_copy(acc.at[pl.ds(s * RT, RT)],
                    out_hbm.at[pl.ds(s * RT, RT), pl.ds(c * DH, DH)])


_BLK = 512
_GRID = N_PAD // _BLK
_full = pl.BlockSpec((_BLK, D), lambda i: (i, 0))
_col = pl.BlockSpec((_BLK, 1), lambda i: (i, 0))


def _tc_mm_body(x_ref, w_ref, deg_ref, o_ref, d_ref):
    dg = deg_ref[...]
    dv = lax.rsqrt(dg[0, :, 0:1] + dg[1, :, 0:1] + 1.0)
    y = lax.dot_general(x_ref[...], w_ref[...], (((1,), (1,)), ((), ())),
                        preferred_element_type=jnp.float32)
    o_ref[...] = y * dv
    d_ref[...] = dv


def _tc_mm(x, w, deg2):
    return pl.pallas_call(
        _tc_mm_body,
        grid=(_GRID,),
        in_specs=[
            _full,
            pl.BlockSpec((D, D), lambda i: (0, 0)),
            pl.BlockSpec((2, _BLK, DW), lambda i: (0, i, 0)),
        ],
        out_specs=[_full, _col],
        out_shape=[jax.ShapeDtypeStruct((N_PAD, D), jnp.float32),
                   jax.ShapeDtypeStruct((N_PAD, 1), jnp.float32)],
    )(x, w, deg2)


def _tc_fuse_body(p_ref, xs_ref, d_ref, b_ref, w_ref, o_ref):
    dv = d_ref[...]
    h = jnp.maximum((p_ref[...] + xs_ref[...]) * dv + b_ref[...], 0.0)
    y = lax.dot_general(h, w_ref[...], (((1,), (1,)), ((), ())),
                        preferred_element_type=jnp.float32)
    o_ref[...] = y * dv


def _tc_fuse(p, xs, dinv, b, w):
    return pl.pallas_call(
        _tc_fuse_body,
        grid=(_GRID,),
        in_specs=[
            _full, _full, _col,
            pl.BlockSpec((1, D), lambda i: (0, 0)),
            pl.BlockSpec((D, D), lambda i: (0, 0)),
        ],
        out_specs=_full,
        out_shape=jax.ShapeDtypeStruct((N_PAD, D), jnp.float32),
    )(p, xs, dinv, b, w)


def _tc_comb_body(p_ref, xs_ref, d_ref, b_ref, o_ref):
    o_ref[...] = jnp.maximum(
        (p_ref[...] + xs_ref[...]) * d_ref[...] + b_ref[...], 0.0)


def _tc_comb(p, xs, dinv, b):
    return pl.pallas_call(
        _tc_comb_body,
        grid=(_GRID,),
        in_specs=[_full, _full, _col,
                  pl.BlockSpec((1, D), lambda i: (0, 0))],
        out_specs=_full,
        out_shape=jax.ShapeDtypeStruct((N_PAD, D), jnp.float32),
    )(p, xs, dinv, b)


def kernel(x, edge_index, W1, b1, W2, b2):
    src = edge_index[0].astype(jnp.int32)
    dst = edge_index[1].astype(jnp.int32)
    x_p = jnp.pad(x, ((0, N_PAD - N_NODES), (0, 0)))
    b1r = b1.reshape(1, D)
    b2r = b2.reshape(1, D)
    s2 = 2 * src
    srcAB = jnp.concatenate([s2, s2 + 1]).reshape(NC * NS, NCH_AGG, CH)
    dstD = dst.reshape(NW, NCH_DEG, CH)
    dstA = dst.reshape(NS * NCH_AGG, CH)

    deg2 = _sc_deg(dstD).reshape(NC, N_PAD, DW)
    xs1, dinv = _tc_mm(x_p, W1, deg2)

    p1 = _sc_agg(xs1.reshape(2 * N_PAD, DH), srcAB, dstA)
    xs2 = _tc_fuse(p1, xs1, dinv, b1r, W2)
    p2 = _sc_agg(xs2.reshape(2 * N_PAD, DH), srcAB, dstA)
    h2 = _tc_comb(p2, xs2, dinv, b2r)
    return h2[:N_NODES]


# speedup vs baseline: 1.5263x; 1.1509x over previous
import functools

import jax
import jax.numpy as jnp
from jax import lax
from jax.experimental import pallas as pl
from jax.experimental.pallas import tpu as pltpu
from jax.experimental.pallas import tpu_sc as plsc

N_NODES = 10000
N_PAD = 10240
N_EDGES = 320000
D = 128
DH = D // 2
NC = 2
NS = 16
NW = NC * NS
CH = 80
NCH_DEG = N_EDGES // NW // CH
NCH_AGG = N_EDGES // NS // CH
RT = N_PAD // NS
DW = 16
NB = 4
NR_DEG = (NCH_DEG - NB) // NB
NR_AGG = (NCH_AGG - NB) // NB

_mesh = plsc.VectorSubcoreMesh(core_axis_name="c", subcore_axis_name="s")


@functools.partial(
    pl.kernel,
    out_type=jax.ShapeDtypeStruct((NC * N_PAD, DW), jnp.float32),
    mesh=_mesh,
    scratch_types=[
        pltpu.VMEM((CH, DW), jnp.float32),
        pltpu.VMEM((CH, DW), jnp.float32),
        pltpu.VMEM_SHARED((N_PAD, DW), jnp.float32),
        pltpu.SemaphoreType.DMA((NB,)),
        pltpu.SemaphoreType.DMA((NB,)),
        pltpu.VMEM((CH,), jnp.int32),
        pltpu.VMEM((CH,), jnp.int32),
        pltpu.VMEM((CH,), jnp.int32),
        pltpu.VMEM((CH,), jnp.int32),
    ],
)
def _sc_deg(dst_hbm, deg_hbm, ones_v, zeros_v, acc, sem_i, sem_s,
            dst_v0, dst_v1, dst_v2, dst_v3):
    dst_v = (dst_v0, dst_v1, dst_v2, dst_v3)
    c = lax.axis_index("c")
    s = lax.axis_index("s")
    w = s * NC + c

    one = jnp.full((16,), 1.0, jnp.float32)
    z = jnp.zeros((16,), jnp.float32)

    def fill(i, _):
        ones_v[i, pl.ds(0, 16)] = one
        zeros_v[i, pl.ds(0, 16)] = z
        return 0

    lax.fori_loop(0, CH, fill, 0)

    def zinit(i, _):
        pltpu.sync_copy(zeros_v, acc.at[pl.ds(s * RT + i * CH, CH)])
        return 0

    lax.fori_loop(0, RT // CH, zinit, 0)

    for j in range(NB):
        pltpu.async_copy(dst_hbm.at[w * NCH_DEG + j], dst_v[j], sem_i.at[j])

    plsc.subcore_barrier()

    def round_body(r, _):
        base = w * NCH_DEG + r * NB
        for j in range(NB):
            pltpu.make_async_copy(dst_hbm.at[base + j], dst_v[j],
                                  sem_i.at[j]).wait()
            pltpu.async_copy(ones_v, acc.at[dst_v[j]], sem_s.at[j], add=True)
        for j in range(NB):
            pltpu.make_async_copy(ones_v, acc.at[dst_v[j]],
                                  sem_s.at[j]).wait()
            pltpu.async_copy(dst_hbm.at[base + NB + j], dst_v[j],
                             sem_i.at[j])
        return 0

    lax.fori_loop(0, NR_DEG, round_body, 0)

    for g in range(NR_DEG * NB, NCH_DEG):
        j = g % NB
        pltpu.make_async_copy(dst_hbm.at[w * NCH_DEG + g], dst_v[j],
                              sem_i.at[j]).wait()
        pltpu.sync_copy(ones_v, acc.at[dst_v[j]], add=True)
        if g + NB < NCH_DEG:
            pltpu.async_copy(dst_hbm.at[w * NCH_DEG + g + NB], dst_v[j],
                             sem_i.at[j])

    plsc.subcore_barrier()

    pltpu.sync_copy(acc.at[pl.ds(s * RT, RT)],
                    deg_hbm.at[pl.ds(c * N_PAD + s * RT, RT)])


@functools.partial(
    pl.kernel,
    out_type=jax.ShapeDtypeStruct((N_PAD, D), jnp.float32),
    mesh=_mesh,
    scratch_types=[
        pltpu.VMEM((NCH_AGG, CH), jnp.int32),
        pltpu.VMEM((NB, CH, DH), jnp.float32),
        pltpu.VMEM_SHARED((N_PAD, DH), jnp.float32),
        pltpu.SemaphoreType.DMA((NB,)),
        pltpu.SemaphoreType.DMA((NB,)),
        pltpu.SemaphoreType.DMA((NB,)),
        pltpu.VMEM((CH,), jnp.int32),
        pltpu.VMEM((CH,), jnp.int32),
        pltpu.VMEM((CH,), jnp.int32),
        pltpu.VMEM((CH,), jnp.int32),
    ],
    compiler_params=pltpu.CompilerParams(use_tc_tiling_on_sc=False),
)
def _sc_agg(xsp_hbm, srcAB_hbm, dst_hbm, out_hbm,
            src_all, rows_v, acc, sem_g, sem_s, sem_i,
            dst_v0, dst_v1, dst_v2, dst_v3):
    dst_v = (dst_v0, dst_v1, dst_v2, dst_v3)
    c = lax.axis_index("c")
    s = lax.axis_index("s")
    w = c * NS + s

    pltpu.sync_copy(srcAB_hbm.at[w], src_all)

    z = jnp.zeros((16,), jnp.float32)

    def zrow(i, _):
        for j in range(DH // 16):
            rows_v[0, i, pl.ds(j * 16, 16)] = z
        return 0

    lax.fori_loop(0, CH, zrow, 0)

    def zinit(i, _):
        pltpu.sync_copy(rows_v.at[0], acc.at[pl.ds(s * RT + i * CH, CH)])
        return 0

    lax.fori_loop(0, RT // CH, zinit, 0)

    for j in range(NB):
        pltpu.async_copy(dst_hbm.at[s * NCH_AGG + j], dst_v[j], sem_i.at[j])
        pltpu.async_copy(xsp_hbm.at[src_all.at[j]], rows_v.at[j],
                         sem_g.at[j])

    plsc.subcore_barrier()

    def round_body(r, _):
        base = r * NB
        for j in range(NB):
            g = base + j
            pltpu.make_async_copy(xsp_hbm.at[src_all.at[g]], rows_v.at[j],
                                  sem_g.at[j]).wait()
            pltpu.make_async_copy(dst_hbm.at[s * NCH_AGG + g],
                                  dst_v[j], sem_i.at[j]).wait()
            pltpu.async_copy(rows_v.at[j], acc.at[dst_v[j]],
                             sem_s.at[j], add=True)
        for j in range(NB):
            g = base + j
            pltpu.make_async_copy(rows_v.at[j], acc.at[dst_v[j]],
                                  sem_s.at[j]).wait()
            pltpu.async_copy(dst_hbm.at[s * NCH_AGG + g + NB], dst_v[j], sem_i.at[j])
            pltpu.async_copy(xsp_hbm.at[src_all.at[g + NB]], rows_v.at[j],
                             sem_g.at[j])
        return 0

    lax.fori_loop(0, NR_AGG, round_body, 0)

    for g in range(NR_AGG * NB, NCH_AGG):
        j = g % NB
        pltpu.make_async_copy(xsp_hbm.at[src_all.at[g]], rows_v.at[j],
                              sem_g.at[j]).wait()
        pltpu.make_async_copy(dst_hbm.at[s * NCH_AGG + g], dst_v[j], sem_i.at[j]).wait()
        pltpu.sync_copy(rows_v.at[j], acc.at[dst_v[j]], add=True)
        if g + NB < NCH_AGG:
            pltpu.async_copy(dst_hbm.at[s * NCH_AGG + g + NB], dst_v[j], sem_i.at[j])
            pltpu.async_copy(xsp_hbm.at[src_all.at[g + NB]], rows_v.at[j],
                             sem_g.at[j])

    plsc.subcore_barrier()

    pltpu.sync_copy(acc.at[pl.ds(s * RT, RT)],
                    out_hbm.at[pl.ds(s * RT, RT), pl.ds(c * DH, DH)])


_BLK = 512
_GRID = N_PAD // _BLK
_full = pl.BlockSpec((_BLK, D), lambda i: (i, 0))
_col = pl.BlockSpec((_BLK, 1), lambda i: (i, 0))


def _tc_mm_body(x_ref, w_ref, deg_ref, o_ref, d_ref):
    dg = deg_ref[...]
    dv = lax.rsqrt(dg[0, :, 0:1] + dg[1, :, 0:1] + 1.0)
    y = lax.dot_general(x_ref[...], w_ref[...], (((1,), (1,)), ((), ())),
                        preferred_element_type=jnp.float32)
    o_ref[...] = y * dv
    d_ref[...] = dv


def _tc_mm(x, w, deg2):
    return pl.pallas_call(
        _tc_mm_body,
        grid=(_GRID,),
        in_specs=[
            _full,
            pl.BlockSpec((D, D), lambda i: (0, 0)),
            pl.BlockSpec((2, _BLK, DW), lambda i: (0, i, 0)),
        ],
        out_specs=[_full, _col],
        out_shape=[jax.ShapeDtypeStruct((N_PAD, D), jnp.float32),
                   jax.ShapeDtypeStruct((N_PAD, 1), jnp.float32)],
    )(x, w, deg2)


def _tc_fuse_body(p_ref, xs_ref, d_ref, b_ref, w_ref, o_ref):
    dv = d_ref[...]
    h = jnp.maximum((p_ref[...] + xs_ref[...]) * dv + b_ref[...], 0.0)
    y = lax.dot_general(h, w_ref[...], (((1,), (1,)), ((), ())),
                        preferred_element_type=jnp.float32)
    o_ref[...] = y * dv


def _tc_fuse(p, xs, dinv, b, w):
    return pl.pallas_call(
        _tc_fuse_body,
        grid=(_GRID,),
        in_specs=[
            _full, _full, _col,
            pl.BlockSpec((1, D), lambda i: (0, 0)),
            pl.BlockSpec((D, D), lambda i: (0, 0)),
        ],
        out_specs=_full,
        out_shape=jax.ShapeDtypeStruct((N_PAD, D), jnp.float32),
    )(p, xs, dinv, b, w)


def _tc_comb_body(p_ref, xs_ref, d_ref, b_ref, o_ref):
    o_ref[...] = jnp.maximum(
        (p_ref[...] + xs_ref[...]) * d_ref[...] + b_ref[...], 0.0)


def _tc_comb(p, xs, dinv, b):
    return pl.pallas_call(
        _tc_comb_body,
        grid=(_GRID,),
        in_specs=[_full, _full, _col,
                  pl.BlockSpec((1, D), lambda i: (0, 0))],
        out_specs=_full,
        out_shape=jax.ShapeDtypeStruct((N_PAD, D), jnp.float32),
    )(p, xs, dinv, b)


def kernel(x, edge_index, W1, b1, W2, b2):
    src = edge_index[0].astype(jnp.int32)
    dst = edge_index[1].astype(jnp.int32)
    x_p = jnp.pad(x, ((0, N_PAD - N_NODES), (0, 0)))
    b1r = b1.reshape(1, D)
    b2r = b2.reshape(1, D)
    s2 = 2 * src
    srcAB = jnp.concatenate([s2, s2 + 1]).reshape(NC * NS, NCH_AGG, CH)
    dstD = dst.reshape(NW * NCH_DEG, CH)
    dstA = dst.reshape(NS * NCH_AGG, CH)

    deg2 = _sc_deg(dstD).reshape(NC, N_PAD, DW)
    xs1, dinv = _tc_mm(x_p, W1, deg2)

    p1 = _sc_agg(xs1.reshape(2 * N_PAD, DH), srcAB, dstA)
    xs2 = _tc_fuse(p1, xs1, dinv, b1r, W2)
    p2 = _sc_agg(xs2.reshape(2 * N_PAD, DH), srcAB, dstA)
    h2 = _tc_comb(p2, xs2, dinv, b2r)
    return h2[:N_NODES]


# speedup vs baseline: 1.5645x; 1.0250x over previous
import functools

import jax
import jax.numpy as jnp
from jax import lax
from jax.experimental import pallas as pl
from jax.experimental.pallas import tpu as pltpu
from jax.experimental.pallas import tpu_sc as plsc

N_NODES = 10000
N_PAD = 10240
N_EDGES = 320000
D = 128
DH = D // 2
NC = 2
NS = 16
NW = NC * NS
CH = 80
CHA = 125
CHZ = 80
NCH_DEG = N_EDGES // NW // CH
NCH_AGG = N_EDGES // NS // CHA
RT = N_PAD // NS
DW = 16
NB = 4
NR_DEG = (NCH_DEG - NB) // NB
NR_AGG = (NCH_AGG - NB) // NB

_mesh = plsc.VectorSubcoreMesh(core_axis_name="c", subcore_axis_name="s")


@functools.partial(
    pl.kernel,
    out_type=jax.ShapeDtypeStruct((NC * N_PAD, DW), jnp.float32),
    mesh=_mesh,
    scratch_types=[
        pltpu.VMEM((CH, DW), jnp.float32),
        pltpu.VMEM((CH, DW), jnp.float32),
        pltpu.VMEM_SHARED((N_PAD, DW), jnp.float32),
        pltpu.SemaphoreType.DMA((NB,)),
        pltpu.SemaphoreType.DMA((NB,)),
        pltpu.VMEM((CH,), jnp.int32),
        pltpu.VMEM((CH,), jnp.int32),
        pltpu.VMEM((CH,), jnp.int32),
        pltpu.VMEM((CH,), jnp.int32),
    ],
)
def _sc_deg(dst_hbm, deg_hbm, ones_v, zeros_v, acc, sem_i, sem_s,
            dst_v0, dst_v1, dst_v2, dst_v3):
    dst_v = (dst_v0, dst_v1, dst_v2, dst_v3)
    c = lax.axis_index("c")
    s = lax.axis_index("s")
    w = s * NC + c

    one = jnp.full((16,), 1.0, jnp.float32)
    z = jnp.zeros((16,), jnp.float32)

    def fill(i, _):
        ones_v[i, pl.ds(0, 16)] = one
        zeros_v[i, pl.ds(0, 16)] = z
        return 0

    lax.fori_loop(0, CH, fill, 0)

    def zinit(i, _):
        pltpu.sync_copy(zeros_v, acc.at[pl.ds(s * RT + i * CH, CH)])
        return 0

    lax.fori_loop(0, RT // CH, zinit, 0)

    for j in range(NB):
        pltpu.async_copy(dst_hbm.at[w * NCH_DEG + j], dst_v[j], sem_i.at[j])

    plsc.subcore_barrier()

    def round_body(r, _):
        base = w * NCH_DEG + r * NB
        for j in range(NB):
            pltpu.make_async_copy(dst_hbm.at[base + j], dst_v[j],
                                  sem_i.at[j]).wait()
            pltpu.async_copy(ones_v, acc.at[dst_v[j]], sem_s.at[j], add=True)
        for j in range(NB):
            pltpu.make_async_copy(ones_v, acc.at[dst_v[j]],
                                  sem_s.at[j]).wait()
            pltpu.async_copy(dst_hbm.at[base + NB + j], dst_v[j],
                             sem_i.at[j])
        return 0

    lax.fori_loop(0, NR_DEG, round_body, 0)

    for g in range(NR_DEG * NB, NCH_DEG):
        j = g % NB
        pltpu.make_async_copy(dst_hbm.at[w * NCH_DEG + g], dst_v[j],
                              sem_i.at[j]).wait()
        pltpu.sync_copy(ones_v, acc.at[dst_v[j]], add=True)
        if g + NB < NCH_DEG:
            pltpu.async_copy(dst_hbm.at[w * NCH_DEG + g + NB], dst_v[j],
                             sem_i.at[j])

    plsc.subcore_barrier()

    pltpu.sync_copy(acc.at[pl.ds(s * RT, RT)],
                    deg_hbm.at[pl.ds(c * N_PAD + s * RT, RT)])


@functools.partial(
    pl.kernel,
    out_type=jax.ShapeDtypeStruct((N_PAD, D), jnp.float32),
    mesh=_mesh,
    scratch_types=[
        pltpu.VMEM((NCH_AGG, CHA), jnp.int32),
        pltpu.VMEM((NB, CHA, DH), jnp.float32),
        pltpu.VMEM_SHARED((N_PAD, DH), jnp.float32),
        pltpu.SemaphoreType.DMA((NB,)),
        pltpu.SemaphoreType.DMA((NB,)),
        pltpu.SemaphoreType.DMA((NB,)),
        pltpu.VMEM((CHA,), jnp.int32),
        pltpu.VMEM((CHA,), jnp.int32),
        pltpu.VMEM((CHA,), jnp.int32),
        pltpu.VMEM((CHA,), jnp.int32),
    ],
    compiler_params=pltpu.CompilerParams(use_tc_tiling_on_sc=False),
)
def _sc_agg(xsp_hbm, srcAB_hbm, dst_hbm, out_hbm,
            src_all, rows_v, acc, sem_g, sem_s, sem_i,
            dst_v0, dst_v1, dst_v2, dst_v3):
    dst_v = (dst_v0, dst_v1, dst_v2, dst_v3)
    c = lax.axis_index("c")
    s = lax.axis_index("s")
    w = c * NS + s

    pltpu.sync_copy(srcAB_hbm.at[w], src_all)

    z = jnp.zeros((16,), jnp.float32)

    def zrow(i, _):
        for j in range(DH // 16):
            rows_v[0, i, pl.ds(j * 16, 16)] = z
        return 0

    lax.fori_loop(0, CHZ, zrow, 0)

    def zinit(i, _):
        pltpu.sync_copy(rows_v.at[0, pl.ds(0, CHZ)],
                        acc.at[pl.ds(s * RT + i * CHZ, CHZ)])
        return 0

    lax.fori_loop(0, RT // CHZ, zinit, 0)

    for j in range(NB):
        pltpu.async_copy(dst_hbm.at[s * NCH_AGG + j], dst_v[j], sem_i.at[j])
        pltpu.async_copy(xsp_hbm.at[src_all.at[j]], rows_v.at[j],
                         sem_g.at[j])

    plsc.subcore_barrier()

    def round_body(r, _):
        base = r * NB
        for j in range(NB):
            g = base + j
            pltpu.make_async_copy(xsp_hbm.at[src_all.at[g]], rows_v.at[j],
                                  sem_g.at[j]).wait()
            pltpu.make_async_copy(dst_hbm.at[s * NCH_AGG + g],
                                  dst_v[j], sem_i.at[j]).wait()
            pltpu.async_copy(rows_v.at[j], acc.at[dst_v[j]],
                             sem_s.at[j], add=True)
        for j in range(NB):
            g = base + j
            pltpu.make_async_copy(rows_v.at[j], acc.at[dst_v[j]],
                                  sem_s.at[j]).wait()
            pltpu.async_copy(dst_hbm.at[s * NCH_AGG + g + NB], dst_v[j], sem_i.at[j])
            pltpu.async_copy(xsp_hbm.at[src_all.at[g + NB]], rows_v.at[j],
                             sem_g.at[j])
        return 0

    lax.fori_loop(0, NR_AGG, round_body, 0)

    for g in range(NR_AGG * NB, NCH_AGG):
        j = g % NB
        pltpu.make_async_copy(xsp_hbm.at[src_all.at[g]], rows_v.at[j],
                              sem_g.at[j]).wait()
        pltpu.make_async_copy(dst_hbm.at[s * NCH_AGG + g], dst_v[j], sem_i.at[j]).wait()
        pltpu.sync_copy(rows_v.at[j], acc.at[dst_v[j]], add=True)
        if g + NB < NCH_AGG:
            pltpu.async_copy(dst_hbm.at[s * NCH_AGG + g + NB], dst_v[j], sem_i.at[j])
            pltpu.async_copy(xsp_hbm.at[src_all.at[g + NB]], rows_v.at[j],
                             sem_g.at[j])

    plsc.subcore_barrier()

    pltpu.sync_copy(acc.at[pl.ds(s * RT, RT)],
                    out_hbm.at[pl.ds(s * RT, RT), pl.ds(c * DH, DH)])


_BLK = 512
_GRID = N_PAD // _BLK
_full = pl.BlockSpec((_BLK, D), lambda i: (i, 0))
_col = pl.BlockSpec((_BLK, 1), lambda i: (i, 0))


def _tc_mm_body(x_ref, w_ref, deg_ref, o_ref, d_ref):
    dg = deg_ref[...]
    dv = lax.rsqrt(dg[0, :, 0:1] + dg[1, :, 0:1] + 1.0)
    y = lax.dot_general(x_ref[...], w_ref[...], (((1,), (1,)), ((), ())),
                        preferred_element_type=jnp.float32)
    o_ref[...] = y * dv
    d_ref[...] = dv


def _tc_mm(x, w, deg2):
    return pl.pallas_call(
        _tc_mm_body,
        grid=(_GRID,),
        in_specs=[
            _full,
            pl.BlockSpec((D, D), lambda i: (0, 0)),
            pl.BlockSpec((2, _BLK, DW), lambda i: (0, i, 0)),
        ],
        out_specs=[_full, _col],
        out_shape=[jax.ShapeDtypeStruct((N_PAD, D), jnp.float32),
                   jax.ShapeDtypeStruct((N_PAD, 1), jnp.float32)],
    )(x, w, deg2)


def _tc_fuse_body(p_ref, xs_ref, d_ref, b_ref, w_ref, o_ref):
    dv = d_ref[...]
    h = jnp.maximum((p_ref[...] + xs_ref[...]) * dv + b_ref[...], 0.0)
    y = lax.dot_general(h, w_ref[...], (((1,), (1,)), ((), ())),
                        preferred_element_type=jnp.float32)
    o_ref[...] = y * dv


def _tc_fuse(p, xs, dinv, b, w):
    return pl.pallas_call(
        _tc_fuse_body,
        grid=(_GRID,),
        in_specs=[
            _full, _full, _col,
            pl.BlockSpec((1, D), lambda i: (0, 0)),
            pl.BlockSpec((D, D), lambda i: (0, 0)),
        ],
        out_specs=_full,
        out_shape=jax.ShapeDtypeStruct((N_PAD, D), jnp.float32),
    )(p, xs, dinv, b, w)


def _tc_comb_body(p_ref, xs_ref, d_ref, b_ref, o_ref):
    o_ref[...] = jnp.maximum(
        (p_ref[...] + xs_ref[...]) * d_ref[...] + b_ref[...], 0.0)


def _tc_comb(p, xs, dinv, b):
    return pl.pallas_call(
        _tc_comb_body,
        grid=(_GRID,),
        in_specs=[_full, _full, _col,
                  pl.BlockSpec((1, D), lambda i: (0, 0))],
        out_specs=_full,
        out_shape=jax.ShapeDtypeStruct((N_PAD, D), jnp.float32),
    )(p, xs, dinv, b)


def kernel(x, edge_index, W1, b1, W2, b2):
    src = edge_index[0].astype(jnp.int32)
    dst = edge_index[1].astype(jnp.int32)
    x_p = jnp.pad(x, ((0, N_PAD - N_NODES), (0, 0)))
    b1r = b1.reshape(1, D)
    b2r = b2.reshape(1, D)
    s2 = 2 * src
    srcAB = jnp.concatenate([s2, s2 + 1]).reshape(NC * NS, NCH_AGG, CHA)
    dstD = dst.reshape(NW * NCH_DEG, CH)
    dstA = dst.reshape(NS * NCH_AGG, CHA)

    deg2 = _sc_deg(dstD).reshape(NC, N_PAD, DW)
    xs1, dinv = _tc_mm(x_p, W1, deg2)

    p1 = _sc_agg(xs1.reshape(2 * N_PAD, DH), srcAB, dstA)
    xs2 = _tc_fuse(p1, xs1, dinv, b1r, W2)
    p2 = _sc_agg(xs2.reshape(2 * N_PAD, DH), srcAB, dstA)
    h2 = _tc_comb(p2, xs2, dinv, b2r)
    return h2[:N_NODES]
